# baseline (speedup 1.0000x reference)
import functools

import jax
import jax.numpy as jnp
from jax import lax
from jax.experimental import pallas as pl
from jax.experimental.pallas import tpu as pltpu
from jax.experimental.pallas import tpu_sc as plsc

B = 4096
L = 8
D = 64
NLANE = 16
NCORE = 2
NSUB = 16
NW = NCORE * NSUB
BPW = B // NW
CHUNK = 128
NCH = (BPW * L) // CHUNK
DV = D // NLANE
GRP = 16
CPW = BPW * L


def _wid():
    return lax.axis_index("s") * NCORE + lax.axis_index("c")


CH = 32


def _stage1_body(uids_hbm, rec_hbm, src_hbm, codes_hbm,
                 uid_v, colv, dst3, codes_v, sem):
    base = _wid() * BPW
    iota = lax.broadcasted_iota(jnp.int32, (NLANE,), 0)

    pltpu.sync_copy(uids_hbm.at[pl.ds(base, BPW)], uid_v)

    def rs_body(c, carry):
        colv[pl.ds(c * NLANE, NLANE)] = uid_v[pl.ds(c * NLANE, NLANE)] & 127
        return carry
    lax.fori_loop(0, BPW // NLANE, rs_body, 0)

    for tab, off in ((rec_hbm, 0), (src_hbm, CPW)):
        for k in range(0, BPW, CH):
            copies = []
            for j in range(CH):
                if j % NLANE == 0:
                    uv = uid_v[pl.ds(k + j, NLANE)]
                ct = pl.multiple_of((uv[j % NLANE] >> 7) * 128, 128)
                copies.append(
                    pltpu.async_copy(tab.at[:, pl.ds(ct, 128)],
                                     dst3.at[j], sem))
            for c in copies:
                c.wait()

            def ex_body(v, carry):
                pv = v * NLANE + iota
                i = pv >> 3
                cc = plsc.load_gather(colv, [k + i])
                cv = plsc.load_gather(dst3, [i, pv & (L - 1), cc])
                codes_v[pl.ds(off + k * L + v * NLANE, NLANE)] = cv
                return carry
            lax.fori_loop(0, (CH * L) // NLANE, ex_body, 0)

    pltpu.sync_copy(codes_v, codes_hbm.at[pl.ds(_wid() * 2 * CPW, 2 * CPW)])


@functools.partial(
    pl.kernel,
    out_type=jax.ShapeDtypeStruct((2 * B * L,), jnp.int32),
    mesh=plsc.VectorSubcoreMesh(core_axis_name="c", subcore_axis_name="s"),
    compiler_params=pltpu.CompilerParams(needs_layout_passes=False,
                                         disable_bounds_checks=True,
                                         skip_device_barrier=True),
    scratch_types=[
        pltpu.VMEM((BPW,), jnp.int32),
        pltpu.VMEM((BPW,), jnp.int32),
        pltpu.VMEM((CH, L, 128), jnp.int32),
        pltpu.VMEM((2 * CPW,), jnp.int32),
        pltpu.SemaphoreType.DMA,
    ],
)
def _stage1(uids_hbm, rec_hbm, src_hbm, codes_hbm,
            uid_v, colv, dst3, codes_v, sem):
    _stage1_body(uids_hbm, rec_hbm, src_hbm, codes_hbm,
                 uid_v, colv, dst3, codes_v, sem)


SEG = 256
NSEG = CPW // SEG
SCH = SEG // CHUNK


def _stage2_body(codes_hbm, emb_hbm, out_hbm,
                 codes_v, ind_v, cnt_v, rows_a, rows_b, out_v, emb0_v, sem):
    wid = _wid()
    base = wid * BPW
    iota = lax.broadcasted_iota(jnp.int32, (NLANE,), 0)

    pltpu.sync_copy(codes_hbm.at[pl.ds(wid * 2 * CPW, 2 * CPW)], codes_v)
    pltpu.sync_copy(emb_hbm.at[pl.ds(0, 8)], emb0_v)

    e0 = [emb0_v[0, pl.ds(d * NLANE, NLANE)] for d in range(DV)]
    bufs = (rows_a, rows_b)

    def fire(seg):
        toff = seg * SEG
        buf = bufs[seg % 2]
        return [
            pltpu.async_copy(emb_hbm.at[codes_v.at[pl.ds(toff + j * CHUNK,
                                                         CHUNK)]],
                             buf.at[pl.ds(j * CHUNK, CHUNK)], sem)
            for j in range(SCH)
        ]

    def fl_body(c, carry):
        cv = codes_v[pl.ds(c * NLANE, NLANE)]
        ind_v[pl.ds(c * NLANE, NLANE)] = jnp.where(
            cv == 0, jnp.float32(1.0), jnp.float32(0.0))
        return carry

    def cnt_body(bc, carry):
        bvec = (bc * NLANE + iota) * L
        acc = plsc.load_gather(ind_v, [bvec])
        for l in range(1, L):
            acc = acc + plsc.load_gather(ind_v, [bvec + l])
        cnt_v[pl.ds(bc * NLANE, NLANE)] = acc
        return carry

    def accum(seg):
        buf = bufs[seg % 2]
        col_base = 0 if seg < NSEG else D
        b0 = (seg % NSEG) * (SEG // L)

        def b_body(b, carry):
            rbase = b * L
            acc = [buf[rbase, pl.ds(d * NLANE, NLANE)] for d in range(DV)]
            for l in range(1, L):
                for d in range(DV):
                    acc[d] = acc[d] + buf[rbase + l, pl.ds(d * NLANE, NLANE)]
            cw = plsc.load_gather(
                cnt_v, [jnp.full((NLANE,), (seg % NSEG) * (SEG // L)
                                 + (0 if seg < NSEG else BPW) + b,
                                 jnp.int32)])
            for d in range(DV):
                out_v[b0 + b, pl.ds(col_base + d * NLANE, NLANE)] = (
                    acc[d] - cw * e0[d])
            return carry
        lax.fori_loop(0, SEG // L, b_body, 0)

    inflight = {}
    inflight[0] = fire(0)
    inflight[1] = fire(1)
    lax.fori_loop(0, (2 * CPW) // NLANE, fl_body, 0)
    lax.fori_loop(0, (2 * BPW) // NLANE, cnt_body, 0)
    for seg in range(2 * NSEG):
        for c in inflight.pop(seg):
            c.wait()
        if seg + 2 < 2 * NSEG:
            inflight[seg + 2] = fire(seg + 2)
        accum(seg)

    pltpu.sync_copy(out_v, out_hbm.at[pl.ds(base, BPW)])


@functools.partial(
    pl.kernel,
    out_type=jax.ShapeDtypeStruct((B, 2 * D), jnp.float32),
    mesh=plsc.VectorSubcoreMesh(core_axis_name="c", subcore_axis_name="s"),
    compiler_params=pltpu.CompilerParams(needs_layout_passes=False,
                                         disable_bounds_checks=True,
                                         skip_device_barrier=True),
    scratch_types=[
        pltpu.VMEM((2 * CPW,), jnp.int32),
        pltpu.VMEM((2 * CPW,), jnp.float32),
        pltpu.VMEM((2 * BPW,), jnp.float32),
        pltpu.VMEM((SEG, 2 * D), jnp.float32),
        pltpu.VMEM((SEG, 2 * D), jnp.float32),
        pltpu.VMEM((BPW, 2 * D), jnp.float32),
        pltpu.VMEM((8, 2 * D), jnp.float32),
        pltpu.SemaphoreType.DMA,
    ],
)
def _stage2(codes_hbm, emb_hbm, out_hbm,
            codes_v, ind_v, cnt_v, rows_a, rows_b, out_v, emb0_v, sem):
    _stage2_body(codes_hbm, emb_hbm, out_hbm,
                 codes_v, ind_v, cnt_v, rows_a, rows_b, out_v, emb0_v, sem)


def kernel(user_ids, user2rec_code, user2src_code, code_embedding):
    codes = _stage1(user_ids, user2rec_code.T, user2src_code.T)
    embp = jnp.concatenate(
        [code_embedding, jnp.zeros_like(code_embedding)], axis=1)
    return _stage2(codes, embp)


# speedup vs baseline: 1.0522x; 1.0522x over previous
import functools

import jax
import jax.numpy as jnp
from jax import lax
from jax.experimental import pallas as pl
from jax.experimental.pallas import tpu as pltpu
from jax.experimental.pallas import tpu_sc as plsc

B = 4096
L = 8
D = 64
NLANE = 16
NCORE = 2
NSUB = 16
NW = NCORE * NSUB
BPW = B // NW
CHUNK = 128
NCH = (BPW * L) // CHUNK
DV = D // NLANE
GRP = 16
CPW = BPW * L


def _wid():
    return lax.axis_index("s") * NCORE + lax.axis_index("c")


CH = 32


def _stage1_body(uids_hbm, rec_hbm, src_hbm, codes_hbm,
                 uid_v, colv, dst3, codes_v, sem):
    base = _wid() * BPW
    iota = lax.broadcasted_iota(jnp.int32, (NLANE,), 0)

    pltpu.sync_copy(uids_hbm.at[pl.ds(base, BPW)], uid_v)

    def rs_body(c, carry):
        colv[pl.ds(c * NLANE, NLANE)] = uid_v[pl.ds(c * NLANE, NLANE)] & 127
        return carry
    lax.fori_loop(0, BPW // NLANE, rs_body, 0)

    chunks = [(tab, off, k)
              for tab, off in ((rec_hbm, 0), (src_hbm, CPW))
              for k in range(0, BPW, CH)]

    def fire(ci, buf):
        tab, off, k = chunks[ci]
        copies = []
        for j in range(CH):
            if j % NLANE == 0:
                uv = uid_v[pl.ds(k + j, NLANE)]
            ct = pl.multiple_of((uv[j % NLANE] >> 7) * 128, 128)
            copies.append(
                pltpu.async_copy(tab.at[:, pl.ds(ct, 128)],
                                 dst3.at[buf * CH + j], sem))
        return copies

    inflight = fire(0, 0)
    for ci in range(len(chunks)):
        _, off, k = chunks[ci]
        buf = ci % 2
        nxt = (fire(ci + 1, 1 - buf) if ci + 1 < len(chunks) else None)
        for c in inflight:
            c.wait()
        inflight = nxt

        def ex_body(v, carry):
            pv = v * NLANE + iota
            i = pv >> 3
            cc = plsc.load_gather(colv, [k + i])
            cv = plsc.load_gather(dst3, [buf * CH + i, pv & (L - 1), cc])
            codes_v[pl.ds(off + k * L + v * NLANE, NLANE)] = cv
            return carry
        lax.fori_loop(0, (CH * L) // NLANE, ex_body, 0)

    pltpu.sync_copy(codes_v, codes_hbm.at[pl.ds(_wid() * 2 * CPW, 2 * CPW)])


@functools.partial(
    pl.kernel,
    out_type=jax.ShapeDtypeStruct((2 * B * L,), jnp.int32),
    mesh=plsc.VectorSubcoreMesh(core_axis_name="c", subcore_axis_name="s"),
    compiler_params=pltpu.CompilerParams(needs_layout_passes=False,
                                         disable_bounds_checks=True,
                                         skip_device_barrier=True),
    scratch_types=[
        pltpu.VMEM((BPW,), jnp.int32),
        pltpu.VMEM((BPW,), jnp.int32),
        pltpu.VMEM((2 * CH, L, 128), jnp.int32),
        pltpu.VMEM((2 * CPW,), jnp.int32),
        pltpu.SemaphoreType.DMA,
    ],
)
def _stage1(uids_hbm, rec_hbm, src_hbm, codes_hbm,
            uid_v, colv, dst3, codes_v, sem):
    _stage1_body(uids_hbm, rec_hbm, src_hbm, codes_hbm,
                 uid_v, colv, dst3, codes_v, sem)


def _stage2_body(codes_hbm, emb_hbm, out_hbm,
                 codes_v, ind_v, cnt_v, rows_v, out_v, emb0_v, sem):
    wid = _wid()
    base = wid * BPW
    iota = lax.broadcasted_iota(jnp.int32, (NLANE,), 0)

    pltpu.sync_copy(codes_hbm.at[pl.ds(wid * 2 * CPW, 2 * CPW)], codes_v)
    pltpu.sync_copy(emb_hbm.at[pl.ds(0, 1)], emb0_v)

    e0 = [emb0_v[0, pl.ds(d * NLANE, NLANE)] for d in range(DV)]

    def do_table(toff, col_base):
        copies = [
            pltpu.async_copy(emb_hbm.at[codes_v.at[pl.ds(toff + j * CHUNK,
                                                         CHUNK)]],
                             rows_v.at[pl.ds(j * CHUNK, CHUNK)], sem)
            for j in range(NCH)
        ]

        def fl_body(c, carry):
            cv = codes_v[pl.ds(toff + c * NLANE, NLANE)]
            ind_v[pl.ds(c * NLANE, NLANE)] = jnp.where(
                cv == 0, jnp.float32(1.0), jnp.float32(0.0))
            return carry
        lax.fori_loop(0, CPW // NLANE, fl_body, 0)

        def cnt_body(bc, carry):
            bvec = (bc * NLANE + iota) * L
            acc = plsc.load_gather(ind_v, [bvec])
            for l in range(1, L):
                acc = acc + plsc.load_gather(ind_v, [bvec + l])
            cnt_v[pl.ds(bc * NLANE, NLANE)] = acc
            return carry
        lax.fori_loop(0, BPW // NLANE, cnt_body, 0)

        for c in copies:
            c.wait()

        def b_body(b, carry):
            rbase = b * L
            acc = [rows_v[rbase, pl.ds(d * NLANE, NLANE)] for d in range(DV)]
            for l in range(1, L):
                for d in range(DV):
                    acc[d] = acc[d] + rows_v[rbase + l, pl.ds(d * NLANE, NLANE)]
            cw = plsc.load_gather(cnt_v, [jnp.full((NLANE,), b, jnp.int32)])
            for d in range(DV):
                out_v[b, pl.ds(col_base + d * NLANE, NLANE)] = acc[d] - cw * e0[d]
            return carry
        lax.fori_loop(0, BPW, b_body, 0)

    do_table(0, 0)
    do_table(CPW, D)

    pltpu.sync_copy(out_v, out_hbm.at[pl.ds(base, BPW)])


@functools.partial(
    pl.kernel,
    out_type=jax.ShapeDtypeStruct((B, 2 * D), jnp.float32),
    mesh=plsc.VectorSubcoreMesh(core_axis_name="c", subcore_axis_name="s"),
    compiler_params=pltpu.CompilerParams(needs_layout_passes=False,
                                         use_tc_tiling_on_sc=False,
                                         disable_bounds_checks=True,
                                         skip_device_barrier=True),
    scratch_types=[
        pltpu.VMEM((2 * CPW,), jnp.int32),
        pltpu.VMEM((CPW,), jnp.float32),
        pltpu.VMEM((BPW,), jnp.float32),
        pltpu.VMEM((CPW, D), jnp.float32),
        pltpu.VMEM((BPW, 2 * D), jnp.float32),
        pltpu.VMEM((1, D), jnp.float32),
        pltpu.SemaphoreType.DMA,
    ],
)
def _stage2(codes_hbm, emb_hbm, out_hbm,
            codes_v, ind_v, cnt_v, rows_v, out_v, emb0_v, sem):
    _stage2_body(codes_hbm, emb_hbm, out_hbm,
                 codes_v, ind_v, cnt_v, rows_v, out_v, emb0_v, sem)


def kernel(user_ids, user2rec_code, user2src_code, code_embedding):
    codes = _stage1(user_ids, user2rec_code.T, user2src_code.T)
    return _stage2(codes, code_embedding)


# speedup vs baseline: 1.0693x; 1.0162x over previous
import functools

import jax
import jax.numpy as jnp
from jax import lax
from jax.experimental import pallas as pl
from jax.experimental.pallas import tpu as pltpu
from jax.experimental.pallas import tpu_sc as plsc

B = 4096
L = 8
D = 64
NLANE = 16
NCORE = 2
NSUB = 16
NW = NCORE * NSUB
BPW = B // NW
CHUNK = 128
NCH = (BPW * L) // CHUNK
DV = D // NLANE
GRP = 16
CPW = BPW * L


def _wid():
    return lax.axis_index("s") * NCORE + lax.axis_index("c")


CH = 32


def _stage1_body(uids_hbm, rec_hbm, src_hbm, codes_hbm,
                 uid_v, colv, dst3, codes_v, sem):
    base = _wid() * BPW
    iota = lax.broadcasted_iota(jnp.int32, (NLANE,), 0)

    pltpu.sync_copy(uids_hbm.at[pl.ds(base, BPW)], uid_v)

    def rs_body(c, carry):
        colv[pl.ds(c * NLANE, NLANE)] = uid_v[pl.ds(c * NLANE, NLANE)] & 127
        return carry
    lax.fori_loop(0, BPW // NLANE, rs_body, 0)

    for tab, off in ((rec_hbm, 0), (src_hbm, CPW)):
        for k in range(0, BPW, CH):
            copies = []
            for j in range(CH):
                if j % NLANE == 0:
                    uv = uid_v[pl.ds(k + j, NLANE)]
                ct = pl.multiple_of((uv[j % NLANE] >> 7) * 128, 128)
                copies.append(
                    pltpu.async_copy(tab.at[:, pl.ds(ct, 128)],
                                     dst3.at[j], sem))
            for c in copies:
                c.wait()

            def ex_body(v, carry):
                pv = v * NLANE + iota
                i = pv >> 3
                cc = plsc.load_gather(colv, [k + i])
                cv = plsc.load_gather(dst3, [i, pv & (L - 1), cc])
                codes_v[pl.ds(off + k * L + v * NLANE, NLANE)] = cv
                return carry
            lax.fori_loop(0, (CH * L) // NLANE, ex_body, 0)

    pltpu.sync_copy(codes_v, codes_hbm.at[pl.ds(_wid() * 2 * CPW, 2 * CPW)])


@functools.partial(
    pl.kernel,
    out_type=jax.ShapeDtypeStruct((2 * B * L,), jnp.int32),
    mesh=plsc.VectorSubcoreMesh(core_axis_name="c", subcore_axis_name="s"),
    compiler_params=pltpu.CompilerParams(needs_layout_passes=False,
                                         disable_bounds_checks=True,
                                         skip_device_barrier=True),
    scratch_types=[
        pltpu.VMEM((BPW,), jnp.int32),
        pltpu.VMEM((BPW,), jnp.int32),
        pltpu.VMEM((CH, L, 128), jnp.int32),
        pltpu.VMEM((2 * CPW,), jnp.int32),
        pltpu.SemaphoreType.DMA,
    ],
)
def _stage1(uids_hbm, rec_hbm, src_hbm, codes_hbm,
            uid_v, colv, dst3, codes_v, sem):
    _stage1_body(uids_hbm, rec_hbm, src_hbm, codes_hbm,
                 uid_v, colv, dst3, codes_v, sem)


def _stage2_body(codes_hbm, emb_hbm, out_hbm,
                 codes_v, ind_v, cnt_v, rows_v, out_v, emb0_v, sem):
    wid = _wid()
    base = wid * BPW
    iota = lax.broadcasted_iota(jnp.int32, (NLANE,), 0)

    pltpu.sync_copy(codes_hbm.at[pl.ds(wid * 2 * CPW, 2 * CPW)], codes_v)
    pltpu.sync_copy(emb_hbm.at[pl.ds(0, 1)], emb0_v)

    e0 = [emb0_v[0, pl.ds(d * NLANE, NLANE)] for d in range(DV)]

    def do_table(toff, col_base):
        copies = [
            pltpu.async_copy(emb_hbm.at[codes_v.at[pl.ds(toff + j * CHUNK,
                                                         CHUNK)]],
                             rows_v.at[pl.ds(j * CHUNK, CHUNK)], sem)
            for j in range(NCH)
        ]

        def fl_body(c, carry):
            cv = codes_v[pl.ds(toff + c * NLANE, NLANE)]
            ind_v[pl.ds(c * NLANE, NLANE)] = jnp.where(
                cv == 0, jnp.float32(1.0), jnp.float32(0.0))
            return carry
        lax.fori_loop(0, CPW // NLANE, fl_body, 0)

        def cnt_body(bc, carry):
            bvec = (bc * NLANE + iota) * L
            acc = plsc.load_gather(ind_v, [bvec])
            for l in range(1, L):
                acc = acc + plsc.load_gather(ind_v, [bvec + l])
            cnt_v[pl.ds(bc * NLANE, NLANE)] = acc
            return carry
        lax.fori_loop(0, BPW // NLANE, cnt_body, 0)

        for c in copies:
            c.wait()

        def b_body(b, carry):
            rbase = b * L
            acc = [rows_v[rbase, pl.ds(d * NLANE, NLANE)] for d in range(DV)]
            for l in range(1, L):
                for d in range(DV):
                    acc[d] = acc[d] + rows_v[rbase + l, pl.ds(d * NLANE, NLANE)]
            cw = plsc.load_gather(cnt_v, [jnp.full((NLANE,), b, jnp.int32)])
            for d in range(DV):
                out_v[b, pl.ds(col_base + d * NLANE, NLANE)] = acc[d] - cw * e0[d]
            return carry
        lax.fori_loop(0, BPW, b_body, 0)

    do_table(0, 0)
    do_table(CPW, D)

    pltpu.sync_copy(out_v, out_hbm.at[pl.ds(base, BPW)])


@functools.partial(
    pl.kernel,
    out_type=jax.ShapeDtypeStruct((B, 2 * D), jnp.float32),
    mesh=plsc.VectorSubcoreMesh(core_axis_name="c", subcore_axis_name="s"),
    compiler_params=pltpu.CompilerParams(needs_layout_passes=False,
                                         use_tc_tiling_on_sc=False,
                                         disable_bounds_checks=True,
                                         skip_device_barrier=True),
    scratch_types=[
        pltpu.VMEM((2 * CPW,), jnp.int32),
        pltpu.VMEM((CPW,), jnp.float32),
        pltpu.VMEM((BPW,), jnp.float32),
        pltpu.VMEM((CPW, D), jnp.float32),
        pltpu.VMEM((BPW, 2 * D), jnp.float32),
        pltpu.VMEM((1, D), jnp.float32),
        pltpu.SemaphoreType.DMA,
    ],
)
def _stage2(codes_hbm, emb_hbm, out_hbm,
            codes_v, ind_v, cnt_v, rows_v, out_v, emb0_v, sem):
    _stage2_body(codes_hbm, emb_hbm, out_hbm,
                 codes_v, ind_v, cnt_v, rows_v, out_v, emb0_v, sem)


def kernel(user_ids, user2rec_code, user2src_code, code_embedding):
    codes = _stage1(user_ids, user2rec_code.T, user2src_code.T)
    return _stage2(codes, code_embedding)
